# Initial kernel scaffold; baseline (speedup 1.0000x reference)
import functools

import jax
import jax.numpy as jnp
from jax import lax
from jax.experimental import pallas as pl
from jax.experimental.pallas import tpu as pltpu
from jax.experimental.pallas import tpu_sc as plsc

_NC = 2
_NS = 16
_NW = _NC * _NS
_G = 128
_NBUF = 4


@functools.lru_cache(maxsize=None)
def _emb_call(ngroups: int, D: int):
    mesh = plsc.VectorSubcoreMesh(core_axis_name="c", subcore_axis_name="s")

    @functools.partial(
        pl.kernel,
        mesh=mesh,
        out_type=jax.ShapeDtypeStruct((_NW, ngroups, _G, D), jnp.float32),
        scratch_types=(
            [pltpu.VMEM((ngroups, _G), jnp.int32)]
            + [pltpu.VMEM((_G, D), jnp.float32) for _ in range(_NBUF)]
            + [pltpu.SemaphoreType.DMA for _ in range(2 * _NBUF)]
        ),
    )
    def run(table, ids, out, idx_v, *rest):
        bufs = rest[:_NBUF]
        gsem = rest[_NBUF:2 * _NBUF]
        osem = rest[2 * _NBUF:]
        wid = lax.axis_index("s") * _NC + lax.axis_index("c")

        pltpu.sync_copy(ids.at[wid], idx_v)

        for b in range(_NBUF):
            pltpu.async_copy(table.at[idx_v.at[b]], bufs[b], gsem[b])

        nj = ngroups // _NBUF

        def step(j, carry):
            for b in range(_NBUF):
                g = j * _NBUF + b
                pltpu.make_async_copy(table.at[idx_v.at[g]], bufs[b], gsem[b]).wait()
                pltpu.async_copy(bufs[b], out.at[wid, g], osem[b])

                @pl.when(j < nj - 1)
                def _():
                    pltpu.make_async_copy(bufs[b], out.at[wid, g], osem[b]).wait()
                    pltpu.async_copy(table.at[idx_v.at[g + _NBUF]], bufs[b], gsem[b])

            return carry

        lax.fori_loop(0, nj, step, 0)

        for b in range(_NBUF):
            g = (nj - 1) * _NBUF + b
            pltpu.make_async_copy(bufs[b], out.at[wid, g], osem[b]).wait()

    return run


def kernel(token_ids, W):
    B, S = token_ids.shape
    V, D = W.shape
    total = B * S
    assert total % (_NW * _G) == 0
    ngroups = total // (_NW * _G)
    ids = token_ids.reshape(_NW, ngroups, _G).astype(jnp.int32)
    out = _emb_call(ngroups, D)(W, ids)
    return out.reshape(B, S, D)


# speedup vs baseline: 1.8793x; 1.8793x over previous
import functools

import jax
import jax.numpy as jnp
from jax import lax
from jax.experimental import pallas as pl
from jax.experimental.pallas import tpu as pltpu
from jax.experimental.pallas import tpu_sc as plsc

_NC = 2
_NS = 16
_NW = _NC * _NS
_G = 128
_NBUF = 4


@functools.lru_cache(maxsize=None)
def _emb_call(ngroups: int, D: int):
    mesh = plsc.VectorSubcoreMesh(core_axis_name="c", subcore_axis_name="s")

    @functools.partial(
        pl.kernel,
        mesh=mesh,
        out_type=jax.ShapeDtypeStruct((_NW, ngroups, _G, D), jnp.float32),
        scratch_types=(
            [pltpu.VMEM((ngroups, _G), jnp.int32)]
            + [pltpu.VMEM((_G, D), jnp.float32) for _ in range(_NBUF)]
            + [pltpu.SemaphoreType.DMA for _ in range(2 * _NBUF)]
        ),
        compiler_params=pltpu.CompilerParams(use_tc_tiling_on_sc=False),
    )
    def run(table, ids, out, idx_v, *rest):
        bufs = rest[:_NBUF]
        gsem = rest[_NBUF:2 * _NBUF]
        osem = rest[2 * _NBUF:]
        wid = lax.axis_index("s") * _NC + lax.axis_index("c")

        pltpu.sync_copy(ids.at[wid], idx_v)

        for b in range(_NBUF):
            pltpu.async_copy(table.at[idx_v.at[b]], bufs[b], gsem[b])

        nj = ngroups // _NBUF

        def step(j, carry):
            for b in range(_NBUF):
                g = j * _NBUF + b
                pltpu.make_async_copy(table.at[idx_v.at[g]], bufs[b], gsem[b]).wait()
                pltpu.async_copy(bufs[b], out.at[wid, g], osem[b])

                @pl.when(j < nj - 1)
                def _():
                    pltpu.make_async_copy(bufs[b], out.at[wid, g], osem[b]).wait()
                    pltpu.async_copy(table.at[idx_v.at[g + _NBUF]], bufs[b], gsem[b])

            return carry

        lax.fori_loop(0, nj, step, 0)

        for b in range(_NBUF):
            g = (nj - 1) * _NBUF + b
            pltpu.make_async_copy(bufs[b], out.at[wid, g], osem[b]).wait()

    return run


def kernel(token_ids, W):
    B, S = token_ids.shape
    V, D = W.shape
    total = B * S
    assert total % (_NW * _G) == 0
    ngroups = total // (_NW * _G)
    ids = token_ids.reshape(_NW, ngroups, _G).astype(jnp.int32)
    out = _emb_call(ngroups, D)(W, ids)
    return out.reshape(B, S, D)
